# baseline (speedup 1.0000x reference)
import functools

import jax
import jax.numpy as jnp
from jax import lax
from jax.experimental import pallas as pl
from jax.experimental.pallas import tpu as pltpu
from jax.experimental.pallas import tpu_sc as plsc

B, T, D, E = 1, 2048, 768, 8
BT = 128
NT = T * 2 // BT + E
NSLOT = NT * BT
NEG = -1e30


def _route_body(x_ref, wg_ref, bg_ref, posk_ref, wt_ref, texp_ref):
    x = x_ref[...]
    logits = jnp.dot(x, wg_ref[...], preferred_element_type=jnp.float32)
    logits = logits + bg_ref[0]
    mx = jnp.max(logits, axis=-1, keepdims=True)
    p = jnp.exp(logits - mx)
    g = p / jnp.sum(p, axis=-1, keepdims=True)
    ecols = lax.broadcasted_iota(jnp.int32, (T, E), 1)
    m1 = jnp.max(g, axis=-1, keepdims=True)
    i1 = jnp.min(jnp.where(g == m1, ecols, E), axis=-1, keepdims=True)
    g2 = jnp.where(ecols == i1, NEG, g)
    m2 = jnp.max(g2, axis=-1, keepdims=True)
    i2 = jnp.min(jnp.where(g2 == m2, ecols, E), axis=-1, keepdims=True)
    denom = jnp.maximum(m1 + m2, 1e-12)
    w1 = m1 / denom
    w2 = m2 / denom
    mask = ((ecols == i1) | (ecols == i2)).astype(jnp.float32)
    s = mask
    sh = 1
    while sh < T:
        top = jnp.zeros((sh, E), jnp.float32)
        s = s + jnp.concatenate([top, s[: T - sh]], axis=0)
        sh *= 2
    exc = s - mask
    cnt = s[T - 1 : T, :]
    cnt_i = cnt.astype(jnp.int32)
    pc = ((cnt_i + (BT - 1)) // BT) * BT
    pcf = pc.astype(jnp.float32)
    er = lax.broadcasted_iota(jnp.int32, (E, E), 0)
    ec = lax.broadcasted_iota(jnp.int32, (E, E), 1)
    ls = (er < ec).astype(jnp.float32)
    off = jnp.dot(pcf, ls, preferred_element_type=jnp.float32)
    total = jnp.sum(pcf)
    offr = off + exc
    pos1 = jnp.sum(jnp.where(ecols == i1, offr, 0.0), axis=-1)
    pos2 = jnp.sum(jnp.where(ecols == i2, offr, 0.0), axis=-1)
    posk = jnp.concatenate(
        [pos1[None, :], pos2[None, :], jnp.zeros((6, T), jnp.float32)], axis=0)
    posk_ref[...] = posk.astype(jnp.int32)
    wfull = jnp.concatenate([w1, w2, jnp.zeros((T, E - 2), jnp.float32)], axis=1)
    wt_ref[...] = wfull
    tbase = (lax.broadcasted_iota(jnp.int32, (NT, 1), 0) * BT).astype(jnp.float32)
    inb = (tbase >= off) & (tbase < off + pcf)
    ecols2 = lax.broadcasted_iota(jnp.int32, (NT, E), 1)
    te = jnp.sum(jnp.where(inb, ecols2, 0), axis=-1)
    te = jnp.where(tbase[:, 0] < total, te, E - 1)
    texp_ref[...] = jnp.broadcast_to(te[None, :], (8, NT))


def _route(x2, Wg, bg2, interpret=False):
    return pl.pallas_call(
        _route_body,
        out_shape=(
            jax.ShapeDtypeStruct((8, T), jnp.int32),
            jax.ShapeDtypeStruct((T, E), jnp.float32),
            jax.ShapeDtypeStruct((8, NT), jnp.int32),
        ),
        interpret=interpret,
    )(x2, Wg, bg2)


TPW = T // 32


def _scatter_xs_body(x_hbm, pos_hbm, xs_hbm, rows_v, idx0_v, idx1_v,
                     so0, so1):
    cid = lax.axis_index("c")
    sid = lax.axis_index("s")
    wid = sid * 2 + cid
    base = wid * TPW
    pltpu.sync_copy(x_hbm.at[pl.ds(base, TPW)], rows_v)
    pltpu.sync_copy(pos_hbm.at[pl.ds(base, TPW)], idx0_v)
    pltpu.sync_copy(pos_hbm.at[pl.ds(T + base, TPW)], idx1_v)
    s0 = pltpu.async_copy(rows_v, xs_hbm.at[idx0_v], so0)
    s1 = pltpu.async_copy(rows_v, xs_hbm.at[idx1_v], so1)
    s0.wait()
    s1.wait()


def _scatter_xs(x2, posflat):
    mesh = plsc.VectorSubcoreMesh(core_axis_name="c", subcore_axis_name="s")
    f = pl.kernel(
        _scatter_xs_body,
        mesh=mesh,
        out_type=jax.ShapeDtypeStruct((NSLOT, D), jnp.float32),
        scratch_types=[
            pltpu.VMEM((TPW, D), jnp.float32),
            pltpu.VMEM((TPW,), jnp.int32),
            pltpu.VMEM((TPW,), jnp.int32),
            pltpu.SemaphoreType.DMA,
            pltpu.SemaphoreType.DMA,
        ],
        name="sc_scatter_xs",
    )
    return f(x2, posflat)


def _make_gather_sc(nrows):
    nw = 32
    rpw = nrows // nw
    half = rpw // 2

    def body(table_hbm, idx_hbm, out_hbm, idx_v, rows0, rows1,
             si0, si1, so0, so1):
        cid = lax.axis_index("c")
        sid = lax.axis_index("s")
        wid = sid * 2 + cid
        base = wid * rpw
        pltpu.sync_copy(idx_hbm.at[pl.ds(base, rpw)], idx_v)
        g0 = pltpu.async_copy(table_hbm.at[idx_v.at[pl.ds(0, half)]],
                              rows0, si0)
        g1 = pltpu.async_copy(table_hbm.at[idx_v.at[pl.ds(half, half)]],
                              rows1, si1)
        g0.wait()
        o0 = pltpu.async_copy(rows0, out_hbm.at[pl.ds(base, half)], so0)
        g1.wait()
        o1 = pltpu.async_copy(rows1, out_hbm.at[pl.ds(base + half, half)], so1)
        o0.wait()
        o1.wait()

    def run(table, idx):
        mesh = plsc.VectorSubcoreMesh(core_axis_name="c", subcore_axis_name="s")
        f = pl.kernel(
            body,
            mesh=mesh,
            out_type=jax.ShapeDtypeStruct((nrows, D), jnp.float32),
            scratch_types=[
                pltpu.VMEM((rpw,), jnp.int32),
                pltpu.VMEM((half, D), jnp.float32),
                pltpu.VMEM((half, D), jnp.float32),
                pltpu.SemaphoreType.DMA,
                pltpu.SemaphoreType.DMA,
                pltpu.SemaphoreType.DMA,
                pltpu.SemaphoreType.DMA,
            ],
            name=f"sc_gather_{nrows}",
        )
        return f(table, idx)

    return run


_gather_ys = _make_gather_sc(2 * T)


def _moe_body(texp_ref, xs_ref, w1_ref, b1_ref, w2_ref, b2_ref, w3_ref,
              b3_ref, ys_ref):
    x = xs_ref[...].astype(jnp.bfloat16)
    h = jnp.dot(x, w1_ref[0], preferred_element_type=jnp.float32) + b1_ref[0, 0]
    h = jnp.where(h > 0, h, 0.01 * h).astype(jnp.bfloat16)
    h = jnp.dot(h, w2_ref[0], preferred_element_type=jnp.float32) + b2_ref[0, 0]
    h = jnp.where(h > 0, h, 0.01 * h).astype(jnp.bfloat16)
    y = jnp.dot(h, w3_ref[0], preferred_element_type=jnp.float32) + b3_ref[0, 0]
    ys_ref[...] = y


def _moe(texp, xs, W1, b1r, W2, b2r, W3, b3r, interpret=False):
    wmap = lambda i, s: (s[i], 0, 0)
    grid_spec = pltpu.PrefetchScalarGridSpec(
        num_scalar_prefetch=1,
        grid=(NT,),
        in_specs=[
            pl.BlockSpec((BT, D), lambda i, s: (i, 0)),
            pl.BlockSpec((1, D, D), wmap),
            pl.BlockSpec((1, 1, D), wmap),
            pl.BlockSpec((1, D, D), wmap),
            pl.BlockSpec((1, 1, D), wmap),
            pl.BlockSpec((1, D, D), wmap),
            pl.BlockSpec((1, 1, D), wmap),
        ],
        out_specs=pl.BlockSpec((BT, D), lambda i, s: (i, 0)),
    )
    return pl.pallas_call(
        _moe_body,
        grid_spec=grid_spec,
        out_shape=jax.ShapeDtypeStruct((NSLOT, D), jnp.float32),
        interpret=interpret,
    )(texp, xs, W1, b1r, W2, b2r, W3, b3r)


def _combine_body(ya_ref, yb_ref, wt_ref, out_ref):
    wa = wt_ref[:, 0:1]
    wb = wt_ref[:, 1:2]
    out_ref[...] = wa * ya_ref[...] + wb * yb_ref[...]


def _combine(yg, wt, interpret=False):
    nt = T // 256
    return pl.pallas_call(
        _combine_body,
        grid=(nt,),
        in_specs=[
            pl.BlockSpec((256, D), lambda i: (i, 0)),
            pl.BlockSpec((256, D), lambda i: (nt + i, 0)),
            pl.BlockSpec((256, E), lambda i: (i, 0)),
        ],
        out_specs=pl.BlockSpec((256, D), lambda i: (i, 0)),
        out_shape=jax.ShapeDtypeStruct((T, D), jnp.float32),
        interpret=interpret,
    )(yg, yg, wt)


@jax.jit
def _run(x2, Wg, bg2, W1, b1r, W2, b2r, W3, b3r):
    posk, wt, texp = _route(x2, Wg, bg2)
    posflat = posk[0:2, :].reshape(2 * T)
    xs = _scatter_xs(x2, posflat)
    ys = _moe(texp[0], xs, W1.astype(jnp.bfloat16), b1r,
              W2.astype(jnp.bfloat16), b2r, W3.astype(jnp.bfloat16), b3r)
    yg = _gather_ys(ys, posflat)
    return _combine(yg, wt)


def kernel(x, topn, Wg, bg, W1, b1, W2, b2, W3, b3):
    del topn
    x2 = x.reshape(T, D)
    bg2 = bg.reshape(1, E)
    b1r = b1.reshape(E, 1, D)
    b2r = b2.reshape(E, 1, D)
    b3r = b3.reshape(E, 1, D)
    out = _run(x2, Wg, bg2, W1, b1r, W2, b2r, W3, b3r)
    return out.reshape(B, T, D)


# speedup vs baseline: 1.5252x; 1.5252x over previous
import functools

import jax
import jax.numpy as jnp
from jax import lax
from jax.experimental import pallas as pl
from jax.experimental.pallas import tpu as pltpu

B, T, D, E = 1, 2048, 768, 8
NEG = -1e30


def _gate_body(x_ref, wg_ref, bg_ref, gs_ref):
    x = x_ref[...]
    logits = jnp.dot(x, wg_ref[...], preferred_element_type=jnp.float32)
    logits = logits + bg_ref[0]
    m = jnp.max(logits, axis=-1, keepdims=True)
    p = jnp.exp(logits - m)
    g = p / jnp.sum(p, axis=-1, keepdims=True)
    ecols = lax.broadcasted_iota(jnp.int32, (T, E), 1)
    i1 = jnp.argmax(g, axis=-1)[:, None]
    oh1 = ecols == i1
    g2 = jnp.where(oh1, NEG, g)
    i2 = jnp.argmax(g2, axis=-1)[:, None]
    mask = oh1 | (ecols == i2)
    gs = jnp.where(mask, g, 0.0)
    denom = jnp.maximum(jnp.sum(gs, axis=-1, keepdims=True), 1e-12)
    gs_ref[...] = gs / denom


def _expert_body(x_ref, gs_ref, w1_ref, b1_ref, w2_ref, b2_ref, w3_ref,
                 b3_ref, out_ref):
    e = pl.program_id(0)
    x = x_ref[...].astype(jnp.bfloat16)
    w1 = w1_ref[0].astype(jnp.bfloat16)
    h = jnp.dot(x, w1, preferred_element_type=jnp.float32) + b1_ref[0, 0]
    h = jnp.where(h > 0, h, 0.01 * h).astype(jnp.bfloat16)
    w2 = w2_ref[0].astype(jnp.bfloat16)
    h = jnp.dot(h, w2, preferred_element_type=jnp.float32) + b2_ref[0, 0]
    h = jnp.where(h > 0, h, 0.01 * h).astype(jnp.bfloat16)
    w3 = w3_ref[0].astype(jnp.bfloat16)
    y = jnp.dot(h, w3, preferred_element_type=jnp.float32) + b3_ref[0, 0]
    onehot = (lax.broadcasted_iota(jnp.int32, (E, 1), 0) == e).astype(jnp.float32)
    gcol = jnp.dot(gs_ref[...], onehot, preferred_element_type=jnp.float32)

    @pl.when(e == 0)
    def _():
        out_ref[...] = gcol * y

    @pl.when(e > 0)
    def _():
        out_ref[...] += gcol * y


@functools.partial(jax.jit, static_argnames=("interpret",))
def _run(x2, Wg, bg2, W1, b1r, W2, b2r, W3, b3r, interpret=False):
    gs = pl.pallas_call(
        _gate_body,
        out_shape=jax.ShapeDtypeStruct((T, E), jnp.float32),
        interpret=interpret,
    )(x2, Wg, bg2)

    full = lambda i: (0, 0)
    out = pl.pallas_call(
        _expert_body,
        grid=(E,),
        in_specs=[
            pl.BlockSpec((T, D), full),
            pl.BlockSpec((T, E), full),
            pl.BlockSpec((1, D, D), lambda i: (i, 0, 0)),
            pl.BlockSpec((1, 1, D), lambda i: (i, 0, 0)),
            pl.BlockSpec((1, D, D), lambda i: (i, 0, 0)),
            pl.BlockSpec((1, 1, D), lambda i: (i, 0, 0)),
            pl.BlockSpec((1, D, D), lambda i: (i, 0, 0)),
            pl.BlockSpec((1, 1, D), lambda i: (i, 0, 0)),
        ],
        out_specs=pl.BlockSpec((T, D), full),
        out_shape=jax.ShapeDtypeStruct((T, D), jnp.float32),
        interpret=interpret,
    )(x2, gs, W1, b1r, W2, b2r, W3, b3r)
    return out


def kernel(x, topn, Wg, bg, W1, b1, W2, b2, W3, b3):
    del topn
    x2 = x.reshape(T, D)
    bg2 = bg.reshape(1, E)
    b1r = b1.reshape(E, 1, D)
    b2r = b2.reshape(E, 1, D)
    b3r = b3.reshape(E, 1, D)
    out = _run(x2, Wg, bg2, W1, b1r, W2, b2r, W3, b3r)
    return out.reshape(B, T, D)
